# baseline (speedup 1.0000x reference)
import functools

import jax
import jax.numpy as jnp
from jax import lax
from jax.experimental import pallas as pl
from jax.experimental.pallas import tpu as pltpu
from jax.experimental.pallas import tpu_sc as plsc

_NC = 2
_NS = 16
_NW = _NC * _NS

_ROWS = 16384
_D = 1024
_RPW = _ROWS // _NW
_C = 16
_NCHUNK = _RPW // _C
_VPR = _D // 16
_SX = 4
_SE = 2

_mesh = plsc.VectorSubcoreMesh(core_axis_name="c", subcore_axis_name="s")


@functools.partial(
    pl.kernel,
    out_type=jax.ShapeDtypeStruct((_ROWS, _D), jnp.float32),
    mesh=_mesh,
    scratch_types=[
        pltpu.VMEM((_RPW,), jnp.int32),
        [pltpu.VMEM((_C, _D), jnp.float32) for _ in range(_SX)],
        [pltpu.VMEM((_C, _D), jnp.float32) for _ in range(_SE)],
        [pltpu.SemaphoreType.DMA for _ in range(_SX)],
        [pltpu.SemaphoreType.DMA for _ in range(_SE)],
        [pltpu.SemaphoreType.DMA for _ in range(_SX)],
    ],
)
def _pe_kernel(x_hbm, idx_hbm, enc_hbm, out_hbm, idx_all, xbufs, ebufs,
               sems_x, sems_e, sems_o):
    wid = lax.axis_index("s") * _NC + lax.axis_index("c")
    base = wid * _RPW

    pltpu.sync_copy(idx_hbm.at[pl.ds(base, _RPW)], idx_all)

    def start_x(g):
        j = g % _SX
        return pltpu.async_copy(
            x_hbm.at[pl.ds(base + g * _C, _C)], xbufs[j], sems_x[j])

    def start_e(g):
        je = g % _SE
        idx_c = idx_all.at[pl.ds(g * _C, _C)]
        return pltpu.async_copy(enc_hbm.at[idx_c], ebufs[je], sems_e[je])

    def start_out(g):
        j = g % _SX
        return pltpu.async_copy(
            xbufs[j], out_hbm.at[pl.ds(base + g * _C, _C)], sems_o[j])

    def add_chunk(g):
        j = g % _SX
        je = g % _SE

        def row(r, c):
            for v in range(_VPR):
                sl = pl.ds(16 * v, 16)
                plsc.addupdate(xbufs[j].at[r, sl], ebufs[je][r, sl])
            return c
        lax.fori_loop(0, _C, row, 0)

    xd = {}
    ed = {}
    od = {}

    for g in (0, 1):
        ed[g] = start_e(g)
        xd[g] = start_x(g)

    for g in range(_NCHUNK):
        if g + 2 < _NCHUNK:
            if g - 2 >= 0:
                od[g - 2].wait()
            xd[g + 2] = start_x(g + 2)
        ed[g].wait()
        xd[g].wait()
        add_chunk(g)
        if g + 2 < _NCHUNK:
            ed[g + 2] = start_e(g + 2)
        od[g] = start_out(g)

    for g in range(_NCHUNK - _SX, _NCHUNK):
        od[g].wait()


def kernel(x, custom_positions, encoding):
    b, s, d = x.shape
    xf = x.reshape(_ROWS, _D)
    idx = custom_positions.reshape(_ROWS)
    out = _pe_kernel(xf, idx, encoding)
    return out.reshape(b, s, d)


# speedup vs baseline: 1.0010x; 1.0010x over previous
import functools

import jax
import jax.numpy as jnp
from jax import lax
from jax.experimental import pallas as pl
from jax.experimental.pallas import tpu as pltpu
from jax.experimental.pallas import tpu_sc as plsc

_NC = 2
_NS = 16
_NW = _NC * _NS

_ROWS = 16384
_D = 1024
_RPW = _ROWS // _NW
_C = 16
_NCHUNK = _RPW // _C
_VPR = _D // 16
_SX = 4
_SE = 3

_mesh = plsc.VectorSubcoreMesh(core_axis_name="c", subcore_axis_name="s")


@functools.partial(
    pl.kernel,
    out_type=jax.ShapeDtypeStruct((_ROWS, _D), jnp.float32),
    mesh=_mesh,
    scratch_types=[
        pltpu.VMEM((_RPW,), jnp.int32),
        [pltpu.VMEM((_C, _D), jnp.float32) for _ in range(_SX)],
        [pltpu.VMEM((_C, _D), jnp.float32) for _ in range(_SE)],
        [pltpu.SemaphoreType.DMA for _ in range(_SX)],
        [pltpu.SemaphoreType.DMA for _ in range(_SE)],
        [pltpu.SemaphoreType.DMA for _ in range(_SX)],
    ],
)
def _pe_kernel(x_hbm, idx_hbm, enc_hbm, out_hbm, idx_all, xbufs, ebufs,
               sems_x, sems_e, sems_o):
    wid = lax.axis_index("s") * _NC + lax.axis_index("c")
    base = wid * _RPW

    pltpu.sync_copy(idx_hbm.at[pl.ds(base, _RPW)], idx_all)

    def start_x(g):
        j = g % _SX
        return pltpu.async_copy(
            x_hbm.at[pl.ds(base + g * _C, _C)], xbufs[j], sems_x[j])

    def start_e(g):
        je = g % _SE
        idx_c = idx_all.at[pl.ds(g * _C, _C)]
        return pltpu.async_copy(enc_hbm.at[idx_c], ebufs[je], sems_e[je])

    def start_out(g):
        j = g % _SX
        return pltpu.async_copy(
            xbufs[j], out_hbm.at[pl.ds(base + g * _C, _C)], sems_o[j])

    def add_chunk(g):
        j = g % _SX
        je = g % _SE

        def row(r, c):
            for v in range(_VPR):
                sl = pl.ds(16 * v, 16)
                plsc.addupdate(xbufs[j].at[r, sl], ebufs[je][r, sl])
            return c
        lax.fori_loop(0, _C, row, 0)

    xd = {}
    ed = {}
    od = {}

    for g in (0, 1):
        ed[g] = start_e(g)
        xd[g] = start_x(g)

    for g in range(_NCHUNK):
        if g + 2 < _NCHUNK:
            if g - 2 >= 0:
                od[g - 2].wait()
            xd[g + 2] = start_x(g + 2)
            ed[g + 2] = start_e(g + 2)
        ed[g].wait()
        xd[g].wait()
        add_chunk(g)
        od[g] = start_out(g)

    for g in range(_NCHUNK - _SX, _NCHUNK):
        od[g].wait()


def kernel(x, custom_positions, encoding):
    b, s, d = x.shape
    xf = x.reshape(_ROWS, _D)
    idx = custom_positions.reshape(_ROWS)
    out = _pe_kernel(xf, idx, encoding)
    return out.reshape(b, s, d)
